# baseline (speedup 1.0000x reference)
import jax
import jax.numpy as jnp
from jax import lax
from jax.experimental import pallas as pl
from jax.experimental.pallas import tpu as pltpu
from jax.experimental.pallas import tpu_sc as plsc

NUM_USERS = 100000
NUM_ITEMS = 1000000
K = 16
H = 20
B = 4096

NC = 2
NS = 16
NW = NC * NS
CH = B // NW
E = CH * H


_CK = 4096
_CS = _CK // 8


def _detile_body(xt_ref, p_ref, out_ref):
  x = xt_ref[...]
  p = p_ref[...]
  acc = jax.lax.dot_general(
      x[:, 0:_CS], p[:, 0:128],
      (((0,), (0,)), ((), ())), preferred_element_type=jnp.float32)
  for s in range(1, 8):
    acc += jax.lax.dot_general(
        x[:, s * _CS:(s + 1) * _CS], p[:, s * 128:(s + 1) * 128],
        (((0,), (0,)), ((), ())), preferred_element_type=jnp.float32)
  out_ref[...] = acc


def _detile(xt, n, p):
  grid = (n + _CK - 1) // _CK
  return pl.pallas_call(
      _detile_body,
      grid=(grid,),
      in_specs=[pl.BlockSpec((K, _CK), lambda i: (0, i)),
                pl.BlockSpec((K, 1024), lambda i: (0, 0))],
      out_specs=pl.BlockSpec((_CS, 128), lambda i: (i, 0)),
      out_shape=jax.ShapeDtypeStruct((grid * _CS, 128), jnp.float32),
      compiler_params=pltpu.CompilerParams(fuse_transposed_lhs_in_matmul=True),
  )(xt, p)


def _selector():
  c = jnp.arange(K)
  p = jnp.zeros((K, 1024), jnp.float32)
  for s in range(8):
    p = p.at[c, s * 128 + s * 16 + c].set(1.0)
  return p


def _sc_body(users_h, items_h, uiflat_h, gu_t, gi_t,
             gu_o, gih_o, gmi_o,
             users_v, items_v, uif, histids, guv, gmiv, ghv,
             sem_u, sem_g, sem_m, sem_h, sem_s):
  wid = lax.axis_index("s") * NC + lax.axis_index("c")
  base = wid * CH

  pltpu.sync_copy(users_h.at[pl.ds(base, CH)], users_v)
  pltpu.sync_copy(items_h.at[pl.ds(base, CH)], items_v)

  def jblk(j, c0):
    def tblk(t, c1):
      u16 = users_v[pl.ds(t * 16, 16)]
      f = u16 * H + j
      flat = (lax.shift_right_logical(f, 7) * 256 + 128
              + jnp.bitwise_and(f, 127))
      uif[pl.ds(j * CH + t * 16, 16)] = flat
      return c1
    return lax.fori_loop(0, CH // 16, tblk, c0, unroll=True)
  lax.fori_loop(0, H, jblk, 0)

  gu_cp = pltpu.async_copy(gu_t.at[users_v], guv, sem_g)
  gmi_cp = pltpu.async_copy(gi_t.at[items_v], gmiv, sem_m)

  pltpu.async_copy(uiflat_h.at[uif], histids, sem_u).wait()
  gh_cp = pltpu.async_copy(gi_t.at[histids], ghv, sem_h)

  gu_cp.wait()
  st1 = pltpu.async_copy(guv, gu_o.at[pl.ds(base, CH)], sem_s)
  gmi_cp.wait()
  st2 = pltpu.async_copy(gmiv, gmi_o.at[pl.ds(base, CH)], sem_s)
  gh_cp.wait()
  st3 = pltpu.async_copy(ghv, gih_o.at[pl.ds(wid * E, E)], sem_s)
  st1.wait()
  st2.wait()
  st3.wait()


NUP = ((NUM_USERS + _CK - 1) // _CK) * _CK
NIP = ((NUM_ITEMS + _CK - 1) // _CK) * _CK


def _sc_gather(users, items, gu_rm, gi_rm, uiflat):
  mesh = plsc.VectorSubcoreMesh(core_axis_name="c", subcore_axis_name="s")
  f = pl.kernel(
      _sc_body,
      out_type=[
          jax.ShapeDtypeStruct((B, K), jnp.float32),
          jax.ShapeDtypeStruct((B * H, K), jnp.float32),
          jax.ShapeDtypeStruct((B, K), jnp.float32),
      ],
      mesh=mesh,
      scratch_types=[
          pltpu.VMEM((CH,), jnp.int32),
          pltpu.VMEM((CH,), jnp.int32),
          pltpu.VMEM((E,), jnp.int32),
          pltpu.VMEM((E,), jnp.int32),
          pltpu.VMEM((CH, K), jnp.float32),
          pltpu.VMEM((CH, K), jnp.float32),
          pltpu.VMEM((E, K), jnp.float32),
          pltpu.SemaphoreType.DMA,
          pltpu.SemaphoreType.DMA,
          pltpu.SemaphoreType.DMA,
          pltpu.SemaphoreType.DMA,
          pltpu.SemaphoreType.DMA,
      ],
      compiler_params=pltpu.CompilerParams(use_tc_tiling_on_sc=False),
  )
  return f(users, items, uiflat, gu_rm, gi_rm)


G = 8
R8 = 8
CR = CH // R8
L = 128


def _leaky(x):
  return jnp.where(x >= 0, x, 0.01 * x)


def _tc_body(gu_ref, gh_ref, gmi_ref, wa_ref, wb_ref, w2_ref, b1_ref, b2_ref,
             ones_ref, xui_ref, gs_ref):
  wa = wa_ref[...]
  wb = wb_ref[...]
  w2 = w2_ref[...]
  b1 = b1_ref[...]
  b2 = b2_ref[...]

  gu = gu_ref[...].reshape(G * CR, L)
  au = jnp.dot(gu, wa, preferred_element_type=jnp.float32)

  gh = gh_ref[...].reshape(G * H * CR, L)
  hi = jnp.dot(gh, wb, preferred_element_type=jnp.float32)
  h1 = (hi.reshape(G, H, CR, L)
        + (au + b1).reshape(G, 1, CR, L))
  hbar = jnp.mean(_leaky(h1), axis=1).reshape(G * CR, L)
  gs = jnp.dot(hbar, w2, preferred_element_type=jnp.float32) + b2

  gmi = gmi_ref[...].reshape(G * CR, L)
  ai = jnp.dot(gmi, wb, preferred_element_type=jnp.float32)
  gl = _leaky(au + ai + b1)
  gui = jnp.dot(gl, w2, preferred_element_type=jnp.float32) + b2

  gs_ref[...] = gs
  xui_ref[...] = jnp.dot(gs * gui, ones_ref[...],
                         preferred_element_type=jnp.float32)


def _tc_mlp(gu_f, gih_f, gmi_f, wa8, wb8, w28, b1t, b2t, ones8):
  grid = NW // G
  wspec = pl.BlockSpec((L, L), lambda i: (0, 0))
  bspec = pl.BlockSpec((1, L), lambda i: (0, 0))
  return pl.pallas_call(
      _tc_body,
      grid=(grid,),
      in_specs=[
          pl.BlockSpec((G, CR, L), lambda i: (i, 0, 0)),
          pl.BlockSpec((G, H, CR, L), lambda i: (i, 0, 0, 0)),
          pl.BlockSpec((G, CR, L), lambda i: (i, 0, 0)),
          wspec, wspec, wspec, bspec, bspec,
          pl.BlockSpec((L, R8), lambda i: (0, 0)),
      ],
      out_specs=[
          pl.BlockSpec((G * CR, R8), lambda i: (i, 0)),
          pl.BlockSpec((G * CR, L), lambda i: (i, 0)),
      ],
      out_shape=[
          jax.ShapeDtypeStruct((B // R8, R8), jnp.float32),
          jax.ShapeDtypeStruct((B // R8, L), jnp.float32),
      ],
  )(gu_f, gih_f, gmi_f, wa8, wb8, w28, b1t, b2t, ones8)


def kernel(users, items, Gu, Gi, W1, b1, W2, b2, ui):
  uiflat = (ui.reshape(2, NUM_USERS * H // 128, 128)
            .transpose(1, 0, 2)
            .reshape(2 * NUM_USERS * H))

  gu_rm = Gu
  gi_rm = Gi

  w1t = W1.T
  eye8 = jnp.eye(R8, dtype=jnp.float32)
  wa8 = jnp.kron(eye8, w1t[:K])
  wb8 = jnp.kron(eye8, w1t[K:])
  w28 = jnp.kron(eye8, W2.T)
  b1t = jnp.tile(b1, R8).reshape(1, L)
  b2t = jnp.tile(b2, R8).reshape(1, L)
  ones8 = jnp.kron(eye8, jnp.ones((K, 1), jnp.float32))

  gu_g, gih, gmi = _sc_gather(users, items, gu_rm, gi_rm, uiflat)
  gu_f = gu_g.reshape(NW, CR, L)
  gih_f = gih.reshape(NW, H, CR, L)
  gmi_f = gmi.reshape(NW, CR, L)
  xui8, gs = _tc_mlp(gu_f, gih_f, gmi_f, wa8, wb8, w28, b1t, b2t, ones8)
  return (xui8.reshape(B), gs.reshape(B, K), gmi)


# speedup vs baseline: 2.2355x; 2.2355x over previous
import jax
import jax.numpy as jnp
import numpy as np
from jax import lax
from jax.experimental import pallas as pl
from jax.experimental.pallas import tpu as pltpu
from jax.experimental.pallas import tpu_sc as plsc

NUM_USERS = 100000
NUM_ITEMS = 1000000
K = 16
H = 20
B = 4096

NC = 2
NS = 16
NW = NC * NS
CH = B // NW
E = CH * H


_CK = 16384
_CS = _CK // 8
_SB = _CS.bit_length() - 1


def _detile_body(xt_ref, p_ref, out_ref):
  x = xt_ref[...]
  p = p_ref[...]
  acc = jax.lax.dot_general(
      x[:, 0:_CS], p[:, 0:128],
      (((0,), (0,)), ((), ())), preferred_element_type=jnp.float32)
  for s in range(1, 8):
    acc += jax.lax.dot_general(
        x[:, s * _CS:(s + 1) * _CS], p[:, s * 128:(s + 1) * 128],
        (((0,), (0,)), ((), ())), preferred_element_type=jnp.float32)
  out_ref[...] = acc


def _detile(xt, n, p):
  grid = (n + _CK - 1) // _CK
  return pl.pallas_call(
      _detile_body,
      grid=(grid,),
      in_specs=[pl.BlockSpec((K, _CK), lambda i: (0, i)),
                pl.BlockSpec((K, 1024), lambda i: (0, 0))],
      out_specs=pl.BlockSpec((_CS, 128), lambda i: (i, 0)),
      out_shape=jax.ShapeDtypeStruct((grid * _CS, 128), jnp.float32),
      compiler_params=pltpu.CompilerParams(fuse_transposed_lhs_in_matmul=True),
  )(xt, p)


def _selector():
  p = np.zeros((K, 1024), np.float32)
  for s in range(8):
    for c in range(K):
      p[c, s * 128 + s * 16 + c] = 1.0
  return jnp.asarray(p)


def _sc_body(users_h, items_h, uiflat_h, gu_t, gi_t,
             gu_o, gih_o, gmi_o,
             users_v, items_v, uif, histids, guv, gmiv, ghv,
             sem_u, sem_g, sem_m, sem_h, sem_s):
  wid = lax.axis_index("s") * NC + lax.axis_index("c")
  base = wid * CH

  pltpu.sync_copy(users_h.at[pl.ds(base, CH)], users_v)
  pltpu.sync_copy(items_h.at[pl.ds(base, CH)], items_v)

  def jblk(j, c0):
    def tblk(t, c1):
      u16 = users_v[pl.ds(t * 16, 16)]
      uif[pl.ds(j * CH + t * 16, 16)] = u16 * H + (NUM_USERS * H + j)
      return c1
    return lax.fori_loop(0, CH // 16, tblk, c0, unroll=True)
  lax.fori_loop(0, H, jblk, 0)

  def remap(buf, nwords):
    def tblk(t, c1):
      sl = pl.ds(t * 16, 16)
      v = buf[sl]
      buf[sl] = (jnp.bitwise_and(v, -_CK)
                 + lax.shift_left(jnp.bitwise_and(v, _CS - 1), 3)
                 + jnp.bitwise_and(lax.shift_right_logical(v, _SB), 7))
      return c1
    lax.fori_loop(0, nwords // 16, tblk, 0, unroll=True)

  remap(users_v, CH)
  remap(items_v, CH)
  gu_cp = pltpu.async_copy(gu_t.at[users_v], guv, sem_g)
  gmi_cp = pltpu.async_copy(gi_t.at[items_v], gmiv, sem_m)

  pltpu.async_copy(uiflat_h.at[uif], histids, sem_u).wait()
  remap(histids, E)
  gh_cp = pltpu.async_copy(gi_t.at[histids], ghv, sem_h)

  gu_cp.wait()
  st1 = pltpu.async_copy(guv, gu_o.at[pl.ds(base, CH)], sem_s)
  gmi_cp.wait()
  st2 = pltpu.async_copy(gmiv, gmi_o.at[pl.ds(base, CH)], sem_s)
  gh_cp.wait()
  st3 = pltpu.async_copy(ghv, gih_o.at[pl.ds(wid * E, E)], sem_s)
  st1.wait()
  st2.wait()
  st3.wait()


NUP = ((NUM_USERS + _CK - 1) // _CK) * _CK
NIP = ((NUM_ITEMS + _CK - 1) // _CK) * _CK


def _sc_gather(users, items, gu_rm, gi_rm, uiflat):
  mesh = plsc.VectorSubcoreMesh(core_axis_name="c", subcore_axis_name="s")
  f = pl.kernel(
      _sc_body,
      out_type=[
          jax.ShapeDtypeStruct((B, K), jnp.float32),
          jax.ShapeDtypeStruct((B * H, K), jnp.float32),
          jax.ShapeDtypeStruct((B, K), jnp.float32),
      ],
      mesh=mesh,
      scratch_types=[
          pltpu.VMEM((CH,), jnp.int32),
          pltpu.VMEM((CH,), jnp.int32),
          pltpu.VMEM((E,), jnp.int32),
          pltpu.VMEM((E,), jnp.int32),
          pltpu.VMEM((CH, K), jnp.float32),
          pltpu.VMEM((CH, K), jnp.float32),
          pltpu.VMEM((E, K), jnp.float32),
          pltpu.SemaphoreType.DMA,
          pltpu.SemaphoreType.DMA,
          pltpu.SemaphoreType.DMA,
          pltpu.SemaphoreType.DMA,
          pltpu.SemaphoreType.DMA,
      ],
      compiler_params=pltpu.CompilerParams(use_tc_tiling_on_sc=False),
  )
  return f(users, items, uiflat, gu_rm, gi_rm)


G = 8
R8 = 8
CR = CH // R8
L = 128


def _leaky(x):
  return jnp.where(x >= 0, x, 0.01 * x)


def _tc_body(gu_ref, gh_ref, gmi_ref, wa_ref, wb_ref, w2_ref, b1_ref, b2_ref,
             ones_ref, xui_ref, gs_ref):
  wa = wa_ref[...]
  wb = wb_ref[...]
  w2 = w2_ref[...]
  b1 = b1_ref[...]
  b2 = b2_ref[...]

  gu = gu_ref[...].reshape(G * CR, L)
  au = jnp.dot(gu, wa, preferred_element_type=jnp.float32)

  gh = gh_ref[...].reshape(G * H * CR, L)
  hi = jnp.dot(gh, wb, preferred_element_type=jnp.float32)
  h1 = (hi.reshape(G, H, CR, L)
        + (au + b1).reshape(G, 1, CR, L))
  hbar = jnp.mean(_leaky(h1), axis=1).reshape(G * CR, L)
  gs = jnp.dot(hbar, w2, preferred_element_type=jnp.float32) + b2

  gmi = gmi_ref[...].reshape(G * CR, L)
  ai = jnp.dot(gmi, wb, preferred_element_type=jnp.float32)
  gl = _leaky(au + ai + b1)
  gui = jnp.dot(gl, w2, preferred_element_type=jnp.float32) + b2

  gs_ref[...] = gs
  xui_ref[...] = jnp.dot(gs * gui, ones_ref[...],
                         preferred_element_type=jnp.float32)


def _tc_mlp(gu_f, gih_f, gmi_f, wa8, wb8, w28, b1t, b2t, ones8):
  grid = NW // G
  wspec = pl.BlockSpec((L, L), lambda i: (0, 0))
  bspec = pl.BlockSpec((1, L), lambda i: (0, 0))
  return pl.pallas_call(
      _tc_body,
      grid=(grid,),
      in_specs=[
          pl.BlockSpec((G, CR, L), lambda i: (i, 0, 0)),
          pl.BlockSpec((G, H, CR, L), lambda i: (i, 0, 0, 0)),
          pl.BlockSpec((G, CR, L), lambda i: (i, 0, 0)),
          wspec, wspec, wspec, bspec, bspec,
          pl.BlockSpec((L, R8), lambda i: (0, 0)),
      ],
      out_specs=[
          pl.BlockSpec((G * CR, R8), lambda i: (i, 0)),
          pl.BlockSpec((G * CR, L), lambda i: (i, 0)),
      ],
      out_shape=[
          jax.ShapeDtypeStruct((B // R8, R8), jnp.float32),
          jax.ShapeDtypeStruct((B // R8, L), jnp.float32),
      ],
  )(gu_f, gih_f, gmi_f, wa8, wb8, w28, b1t, b2t, ones8)


def kernel(users, items, Gu, Gi, W1, b1, W2, b2, ui):
  uiflat = ui.reshape(2 * NUM_USERS * H)

  p = _selector()
  gu_rm = _detile(Gu.T, NUM_USERS, p).reshape(NUP, K)
  gi_rm = _detile(Gi.T, NUM_ITEMS, p).reshape(NIP, K)

  w1t = W1.T
  eye8 = jnp.eye(R8, dtype=jnp.float32)
  wa8 = jnp.kron(eye8, w1t[:K])
  wb8 = jnp.kron(eye8, w1t[K:])
  w28 = jnp.kron(eye8, W2.T)
  b1t = jnp.tile(b1, R8).reshape(1, L)
  b2t = jnp.tile(b2, R8).reshape(1, L)
  ones8 = jnp.kron(eye8, jnp.ones((K, 1), jnp.float32))

  gu_g, gih, gmi = _sc_gather(users, items, gu_rm, gi_rm, uiflat)
  gu_f = gu_g.reshape(NW, CR, L)
  gih_f = gih.reshape(NW, H, CR, L)
  gmi_f = gmi.reshape(NW, CR, L)
  xui8, gs = _tc_mlp(gu_f, gih_f, gmi_f, wa8, wb8, w28, b1t, b2t, ones8)
  return (xui8.reshape(B), gs.reshape(B, K), gmi)


# speedup vs baseline: 2.2365x; 1.0005x over previous
import jax
import jax.numpy as jnp
import numpy as np
from jax import lax
from jax.experimental import pallas as pl
from jax.experimental.pallas import tpu as pltpu
from jax.experimental.pallas import tpu_sc as plsc

NUM_USERS = 100000
NUM_ITEMS = 1000000
K = 16
H = 20
B = 4096

NC = 2
NS = 16
NW = NC * NS
CH = B // NW
E = CH * H


_CK = 16384
_CS = _CK // 8
_SB = _CS.bit_length() - 1


def _detile_body(xt_ref, p_ref, out_ref):
  x = xt_ref[...]
  p = p_ref[...]
  acc = jax.lax.dot_general(
      x[:, 0:_CS], p[:, 0:128],
      (((0,), (0,)), ((), ())), preferred_element_type=jnp.float32)
  for s in range(1, 8):
    acc += jax.lax.dot_general(
        x[:, s * _CS:(s + 1) * _CS], p[:, s * 128:(s + 1) * 128],
        (((0,), (0,)), ((), ())), preferred_element_type=jnp.float32)
  out_ref[...] = acc


def _detile(xt, n, p):
  grid = (n + _CK - 1) // _CK
  return pl.pallas_call(
      _detile_body,
      grid=(grid,),
      in_specs=[pl.BlockSpec((K, _CK), lambda i: (0, i)),
                pl.BlockSpec((K, 1024), lambda i: (0, 0))],
      out_specs=pl.BlockSpec((_CS, 128), lambda i: (i, 0)),
      out_shape=jax.ShapeDtypeStruct((grid * _CS, 128), jnp.float32),
      compiler_params=pltpu.CompilerParams(fuse_transposed_lhs_in_matmul=True),
  )(xt, p)


def _selector():
  p = np.zeros((K, 1024), np.float32)
  for s in range(8):
    for c in range(K):
      p[c, s * 128 + s * 16 + c] = 1.0
  return jnp.asarray(p)


def _sc_body(users_h, items_h, uiflat_h, gu_t, gi_t,
             gu_o, gih_o, gmi_o,
             users_v, items_v, uif, histids, guv, gmiv, ghv,
             sem_u, sem_g, sem_m, sem_h, sem_s):
  wid = lax.axis_index("s") * NC + lax.axis_index("c")
  base = wid * CH

  pltpu.sync_copy(users_h.at[pl.ds(base, CH)], users_v)
  pltpu.sync_copy(items_h.at[pl.ds(base, CH)], items_v)

  def jblk(j, c0):
    def tblk(t, c1):
      u16 = users_v[pl.ds(t * 16, 16)]
      uif[pl.ds(j * CH + t * 16, 16)] = u16 * H + j
      return c1
    return lax.fori_loop(0, CH // 16, tblk, c0, unroll=True)
  lax.fori_loop(0, H, jblk, 0)

  def remap(buf, nwords):
    def tblk(t, c1):
      sl = pl.ds(t * 16, 16)
      v = buf[sl]
      buf[sl] = (jnp.bitwise_and(v, -_CK)
                 + lax.shift_left(jnp.bitwise_and(v, _CS - 1), 3)
                 + jnp.bitwise_and(lax.shift_right_logical(v, _SB), 7))
      return c1
    lax.fori_loop(0, nwords // 16, tblk, 0, unroll=True)

  remap(users_v, CH)
  remap(items_v, CH)
  gu_cp = pltpu.async_copy(gu_t.at[users_v], guv, sem_g)
  gmi_cp = pltpu.async_copy(gi_t.at[items_v], gmiv, sem_m)

  pltpu.async_copy(uiflat_h.at[1].at[uif], histids, sem_u).wait()
  remap(histids, E)
  gh_cp = pltpu.async_copy(gi_t.at[histids], ghv, sem_h)

  gu_cp.wait()
  st1 = pltpu.async_copy(guv, gu_o.at[pl.ds(base, CH)], sem_s)
  gmi_cp.wait()
  st2 = pltpu.async_copy(gmiv, gmi_o.at[pl.ds(base, CH)], sem_s)
  gh_cp.wait()
  st3 = pltpu.async_copy(ghv, gih_o.at[pl.ds(wid * E, E)], sem_s)
  st1.wait()
  st2.wait()
  st3.wait()


NUP = ((NUM_USERS + _CK - 1) // _CK) * _CK
NIP = ((NUM_ITEMS + _CK - 1) // _CK) * _CK


def _sc_gather(users, items, gu_rm, gi_rm, uiflat):
  mesh = plsc.VectorSubcoreMesh(core_axis_name="c", subcore_axis_name="s")
  f = pl.kernel(
      _sc_body,
      out_type=[
          jax.ShapeDtypeStruct((B, K), jnp.float32),
          jax.ShapeDtypeStruct((B * H, K), jnp.float32),
          jax.ShapeDtypeStruct((B, K), jnp.float32),
      ],
      mesh=mesh,
      scratch_types=[
          pltpu.VMEM((CH,), jnp.int32),
          pltpu.VMEM((CH,), jnp.int32),
          pltpu.VMEM((E,), jnp.int32),
          pltpu.VMEM((E,), jnp.int32),
          pltpu.VMEM((CH, K), jnp.float32),
          pltpu.VMEM((CH, K), jnp.float32),
          pltpu.VMEM((E, K), jnp.float32),
          pltpu.SemaphoreType.DMA,
          pltpu.SemaphoreType.DMA,
          pltpu.SemaphoreType.DMA,
          pltpu.SemaphoreType.DMA,
          pltpu.SemaphoreType.DMA,
      ],
      compiler_params=pltpu.CompilerParams(use_tc_tiling_on_sc=False),
  )
  return f(users, items, uiflat, gu_rm, gi_rm)


G = 8
R8 = 8
CR = CH // R8
L = 128


def _leaky(x):
  return jnp.where(x >= 0, x, 0.01 * x)


def _tc_body(gu_ref, gh_ref, gmi_ref, wa_ref, wb_ref, w2_ref, b1_ref, b2_ref,
             ones_ref, xui_ref, gs_ref):
  wa = wa_ref[...]
  wb = wb_ref[...]
  w2 = w2_ref[...]
  b1 = b1_ref[...]
  b2 = b2_ref[...]

  gu = gu_ref[...].reshape(G * CR, L)
  au = jnp.dot(gu, wa, preferred_element_type=jnp.float32)

  gh = gh_ref[...].reshape(G * H * CR, L)
  hi = jnp.dot(gh, wb, preferred_element_type=jnp.float32)
  h1 = (hi.reshape(G, H, CR, L)
        + (au + b1).reshape(G, 1, CR, L))
  hbar = jnp.mean(_leaky(h1), axis=1).reshape(G * CR, L)
  gs = jnp.dot(hbar, w2, preferred_element_type=jnp.float32) + b2

  gmi = gmi_ref[...].reshape(G * CR, L)
  ai = jnp.dot(gmi, wb, preferred_element_type=jnp.float32)
  gl = _leaky(au + ai + b1)
  gui = jnp.dot(gl, w2, preferred_element_type=jnp.float32) + b2

  gs_ref[...] = gs
  xui_ref[...] = jnp.dot(gs * gui, ones_ref[...],
                         preferred_element_type=jnp.float32)


def _tc_mlp(gu_f, gih_f, gmi_f, wa8, wb8, w28, b1t, b2t, ones8):
  grid = NW // G
  wspec = pl.BlockSpec((L, L), lambda i: (0, 0))
  bspec = pl.BlockSpec((1, L), lambda i: (0, 0))
  return pl.pallas_call(
      _tc_body,
      grid=(grid,),
      in_specs=[
          pl.BlockSpec((G, CR, L), lambda i: (i, 0, 0)),
          pl.BlockSpec((G, H, CR, L), lambda i: (i, 0, 0, 0)),
          pl.BlockSpec((G, CR, L), lambda i: (i, 0, 0)),
          wspec, wspec, wspec, bspec, bspec,
          pl.BlockSpec((L, R8), lambda i: (0, 0)),
      ],
      out_specs=[
          pl.BlockSpec((G * CR, R8), lambda i: (i, 0)),
          pl.BlockSpec((G * CR, L), lambda i: (i, 0)),
      ],
      out_shape=[
          jax.ShapeDtypeStruct((B // R8, R8), jnp.float32),
          jax.ShapeDtypeStruct((B // R8, L), jnp.float32),
      ],
  )(gu_f, gih_f, gmi_f, wa8, wb8, w28, b1t, b2t, ones8)


def kernel(users, items, Gu, Gi, W1, b1, W2, b2, ui):
  uiflat = ui

  p = _selector()
  gu_rm = _detile(Gu.T, NUM_USERS, p).reshape(NUP, K)
  gi_rm = _detile(Gi.T, NUM_ITEMS, p).reshape(NIP, K)

  w1t = W1.T
  eye8 = jnp.eye(R8, dtype=jnp.float32)
  wa8 = jnp.kron(eye8, w1t[:K])
  wb8 = jnp.kron(eye8, w1t[K:])
  w28 = jnp.kron(eye8, W2.T)
  b1t = jnp.tile(b1, R8).reshape(1, L)
  b2t = jnp.tile(b2, R8).reshape(1, L)
  ones8 = jnp.kron(eye8, jnp.ones((K, 1), jnp.float32))

  gu_g, gih, gmi = _sc_gather(users, items, gu_rm, gi_rm, uiflat)
  gu_f = gu_g.reshape(NW, CR, L)
  gih_f = gih.reshape(NW, H, CR, L)
  gmi_f = gmi.reshape(NW, CR, L)
  xui8, gs = _tc_mlp(gu_f, gih_f, gmi_f, wa8, wb8, w28, b1t, b2t, ones8)
  return (xui8.reshape(B), gs.reshape(B, K), gmi)
